# baseline (speedup 1.0000x reference)
import functools

import jax
import jax.numpy as jnp
from jax import lax
from jax.experimental import pallas as pl
from jax.experimental.pallas import tpu as pltpu
from jax.experimental.pallas import tpu_sc as plsc

B = 16384
F = 64
N_ROWS = 1000000

_info = plsc.get_sparse_core_info()
NC = _info.num_cores
NS = _info.num_subcores
L = _info.num_lanes
NW = NC * NS
BPW = B // NW
CH = 128
NCH = BPW // CH

_mesh = plsc.VectorSubcoreMesh(core_axis_name="c", subcore_axis_name="s")


@functools.partial(
    pl.kernel,
    mesh=_mesh,
    compiler_params=pltpu.CompilerParams(needs_layout_passes=False),
    out_type=jax.ShapeDtypeStruct((B,), jnp.float32),
    scratch_types=[
        pltpu.VMEM((BPW,), jnp.int32),
        pltpu.VMEM((BPW,), jnp.int32),
        pltpu.VMEM((NCH, CH), jnp.int32),
        pltpu.VMEM((NCH, CH), jnp.int32),
        pltpu.VMEM((2, CH, 128), jnp.float32),
        pltpu.VMEM((2, CH, 128), jnp.float32),
        pltpu.VMEM((BPW,), jnp.float32),
        pltpu.SemaphoreType.DMA,
        pltpu.SemaphoreType.DMA,
    ],
)
def _mf_sc(user_hbm, item_hbm, uf2_hbm, if2_hbm, out_hbm,
           uh, ih, upr, ipr, ue, ie, outv, sem0, sem1):
    wid = lax.axis_index("s") * NC + lax.axis_index("c")

    pltpu.sync_copy(user_hbm.at[wid], uh)
    pltpu.sync_copy(item_hbm.at[wid], ih)

    def base_body(j, _):
        sl = pl.ds(j * L, L)
        k = j // (CH // L)
        p = j % (CH // L)
        csl = pl.ds(p * L, L)
        r = uh[sl]
        upr[k, csl] = r >> 1
        uh[sl] = (r & 1) << 6
        r = ih[sl]
        ipr[k, csl] = r >> 1
        ih[sl] = (r & 1) << 6
        return 0

    lax.fori_loop(0, BPW // L, base_body, 0)

    sems = (sem0, sem1)

    def fetch(k, buf):
        pltpu.async_copy(uf2_hbm.at[upr.at[k]], ue.at[buf], sems[buf])
        pltpu.async_copy(if2_hbm.at[ipr.at[k]], ie.at[buf], sems[buf])

    def drain(k, buf):
        pltpu.make_async_copy(uf2_hbm.at[upr.at[k]], ue.at[buf], sems[buf]).wait()
        pltpu.make_async_copy(if2_hbm.at[ipr.at[k]], ie.at[buf], sems[buf]).wait()

    lanes = lax.iota(jnp.int32, L)

    fetch(0, 0)
    for k in range(NCH):
        buf = k % 2
        drain(k, buf)
        if k + 1 < NCH:
            fetch(k + 1, (k + 1) % 2)

        def group_body(p, _, k=k, buf=buf):
            rows = p * L + lanes
            bsl = pl.ds(k * CH + p * L, L)
            hu = uh[bsl]
            hi = ih[bsl]
            acc = jnp.zeros((L,), jnp.float32)
            for j in range(F):
                jv = jnp.full((L,), j, jnp.int32)
                u16 = plsc.load_gather(ue.at[buf], [rows, hu + jv])
                v16 = plsc.load_gather(ie.at[buf], [rows, hi + jv])
                acc = acc + u16 * v16
            outv[bsl] = acc
            return 0

        lax.fori_loop(0, CH // L, group_body, 0)

    pltpu.sync_copy(outv, out_hbm.at[pl.ds(wid * BPW, BPW)])


def kernel(user, item, user_factors, item_factors):
    user_r = user.astype(jnp.int32).reshape(NW, BPW)
    item_r = item.astype(jnp.int32).reshape(NW, BPW)
    one = 1.0 + jnp.minimum(user[0], 0).astype(jnp.float32)
    uf2 = (user_factors * one).reshape(N_ROWS // 2, 2 * F)
    if2 = (item_factors * one).reshape(N_ROWS // 2, 2 * F)
    return _mf_sc(user_r, item_r, uf2, if2)


# speedup vs baseline: 1.5377x; 1.5377x over previous
import functools

import jax
import jax.numpy as jnp
from jax import lax
from jax.experimental import pallas as pl
from jax.experimental.pallas import tpu as pltpu
from jax.experimental.pallas import tpu_sc as plsc

B = 16384
F = 64
N_ROWS = 1000000

_info = plsc.get_sparse_core_info()
NC = _info.num_cores
NS = _info.num_subcores
L = _info.num_lanes
NCK = 1
NW = NCK * NS
BPW = B // NW
CH = 128
NCH = BPW // CH

_mesh = plsc.VectorSubcoreMesh(
    core_axis_name="c", subcore_axis_name="s", num_cores=NCK)


@functools.partial(
    pl.kernel,
    mesh=_mesh,
    compiler_params=pltpu.CompilerParams(needs_layout_passes=False),
    out_type=jax.ShapeDtypeStruct((B,), jnp.float32),
    scratch_types=[
        pltpu.VMEM((BPW,), jnp.int32),
        pltpu.VMEM((BPW,), jnp.int32),
        pltpu.VMEM((NCH, CH), jnp.int32),
        pltpu.VMEM((NCH, CH), jnp.int32),
        pltpu.VMEM((2, CH, 128), jnp.float32),
        pltpu.VMEM((2, CH, 128), jnp.float32),
        pltpu.VMEM((BPW,), jnp.float32),
        pltpu.SemaphoreType.DMA,
        pltpu.SemaphoreType.DMA,
    ],
)
def _mf_sc(user_hbm, item_hbm, uf2_hbm, if2_hbm, out_hbm,
           uh, ih, upr, ipr, ue, ie, outv, sem0, sem1):
    wid = lax.axis_index("s") * NCK + lax.axis_index("c")

    pltpu.sync_copy(user_hbm.at[wid], uh)
    pltpu.sync_copy(item_hbm.at[wid], ih)

    def base_body(j, _):
        sl = pl.ds(j * L, L)
        k = j // (CH // L)
        p = j % (CH // L)
        csl = pl.ds(p * L, L)
        r = uh[sl]
        upr[k, csl] = r >> 1
        uh[sl] = (r & 1) << 6
        r = ih[sl]
        ipr[k, csl] = r >> 1
        ih[sl] = (r & 1) << 6
        return 0

    lax.fori_loop(0, BPW // L, base_body, 0)

    sems = (sem0, sem1)

    def fetch(k, buf):
        pltpu.async_copy(uf2_hbm.at[upr.at[k]], ue.at[buf], sems[buf])
        pltpu.async_copy(if2_hbm.at[ipr.at[k]], ie.at[buf], sems[buf])

    def drain(k, buf):
        pltpu.make_async_copy(uf2_hbm.at[upr.at[k]], ue.at[buf], sems[buf]).wait()
        pltpu.make_async_copy(if2_hbm.at[ipr.at[k]], ie.at[buf], sems[buf]).wait()

    lanes = lax.iota(jnp.int32, L)

    def accum(k, buf):
        def group_body(p, _):
            rows = p * L + lanes
            bsl = pl.ds(k * CH + p * L, L)
            hu = uh[bsl]
            hi = ih[bsl]
            acc = jnp.zeros((L,), jnp.float32)
            for j in range(F):
                jv = jnp.full((L,), j, jnp.int32)
                u16 = plsc.load_gather(ue.at[buf], [rows, hu + jv])
                v16 = plsc.load_gather(ie.at[buf], [rows, hi + jv])
                acc = acc + u16 * v16
            outv[bsl] = acc
            return 0

        lax.fori_loop(0, CH // L, group_body, 0)

    fetch(0, 0)

    def pair_body(h, _):
        k0 = h * 2
        drain(k0, 0)
        fetch(k0 + 1, 1)
        accum(k0, 0)
        drain(k0 + 1, 1)

        @pl.when(k0 + 2 < NCH)
        def _():
            fetch(k0 + 2, 0)

        accum(k0 + 1, 1)
        return 0

    lax.fori_loop(0, NCH // 2, pair_body, 0)

    pltpu.sync_copy(outv, out_hbm.at[pl.ds(wid * BPW, BPW)])


def kernel(user, item, user_factors, item_factors):
    user_r = user.astype(jnp.int32).reshape(NW, BPW)
    item_r = item.astype(jnp.int32).reshape(NW, BPW)
    uf2 = user_factors.reshape(N_ROWS // 2, 2 * F)
    if2 = item_factors.reshape(N_ROWS // 2, 2 * F)
    return _mf_sc(user_r, item_r, uf2, if2)


# speedup vs baseline: 1.5919x; 1.0352x over previous
import functools

import jax
import jax.numpy as jnp
from jax import lax
from jax.experimental import pallas as pl
from jax.experimental.pallas import tpu as pltpu
from jax.experimental.pallas import tpu_sc as plsc

B = 16384
F = 64
N_ROWS = 1000000

_info = plsc.get_sparse_core_info()
NC = _info.num_cores
NS = _info.num_subcores
L = _info.num_lanes
NW = NC * NS
BPW = B // NW
CH = 128
NCH = BPW // CH

_mesh = plsc.VectorSubcoreMesh(core_axis_name="c", subcore_axis_name="s")


@functools.partial(
    pl.kernel,
    mesh=_mesh,
    compiler_params=pltpu.CompilerParams(needs_layout_passes=False),
    out_type=jax.ShapeDtypeStruct((B, 2 * F), jnp.float32),
    scratch_types=[
        pltpu.VMEM((BPW,), jnp.int32),
        pltpu.VMEM((NCH, CH), jnp.int32),
        pltpu.VMEM((2, CH, 2 * F), jnp.float32),
        pltpu.SemaphoreType.DMA,
        pltpu.SemaphoreType.DMA,
    ],
)
def _gather_sc(idx_hbm, tab_hbm, out_hbm, stage, pr, rows, sem0, sem1):
    wid = lax.axis_index("s") * NC + lax.axis_index("c")

    pltpu.sync_copy(idx_hbm.at[wid], stage)

    def base_body(j, _):
        k = j // (CH // L)
        p = j % (CH // L)
        pr[k, pl.ds(p * L, L)] = stage[pl.ds(j * L, L)] >> 1
        return 0

    lax.fori_loop(0, BPW // L, base_body, 0)

    sems = (sem0, sem1)

    def fetch(k, buf):
        pltpu.async_copy(tab_hbm.at[pr.at[k]], rows.at[buf], sems[buf])

    def drain(k, buf):
        pltpu.make_async_copy(
            tab_hbm.at[pr.at[k]], rows.at[buf], sems[buf]).wait()

    fetch(0, 0)
    for k in range(NCH):
        buf = k % 2
        drain(k, buf)
        if k + 1 < NCH:
            fetch(k + 1, (k + 1) % 2)
        pltpu.sync_copy(
            rows.at[buf], out_hbm.at[pl.ds(wid * BPW + k * CH, CH)])


def _dot_body(u_ref, i_ref, pu_ref, pi_ref, o_ref):
    up = u_ref[0, 0, :]
    ip = i_ref[0, 0, :]
    pu = pu_ref[...]
    pi = pi_ref[...]
    su = jnp.where((up & 1)[:, None] == 1, pu[:, F:], pu[:, :F])
    si = jnp.where((ip & 1)[:, None] == 1, pi[:, F:], pi[:, :F])
    o_ref[0, 0, :] = jnp.sum(su * si, axis=1)


_DOT_BLK = 512


def _dot_tc(user3, item3, pairs_u, pairs_i):
    nblk = B // _DOT_BLK
    return pl.pallas_call(
        _dot_body,
        grid=(nblk,),
        in_specs=[
            pl.BlockSpec((1, 1, _DOT_BLK), lambda i: (i, 0, 0)),
            pl.BlockSpec((1, 1, _DOT_BLK), lambda i: (i, 0, 0)),
            pl.BlockSpec((_DOT_BLK, 2 * F), lambda i: (i, 0)),
            pl.BlockSpec((_DOT_BLK, 2 * F), lambda i: (i, 0)),
        ],
        out_specs=pl.BlockSpec((1, 1, _DOT_BLK), lambda i: (i, 0, 0)),
        out_shape=jax.ShapeDtypeStruct((nblk, 1, _DOT_BLK), jnp.float32),
    )(user3, item3, pairs_u, pairs_i)


def kernel(user, item, user_factors, item_factors):
    user_r = user.astype(jnp.int32).reshape(NW, BPW)
    item_r = item.astype(jnp.int32).reshape(NW, BPW)
    uf2 = user_factors.reshape(N_ROWS // 2, 2 * F)
    if2 = item_factors.reshape(N_ROWS // 2, 2 * F)
    pairs_u = _gather_sc(user_r, uf2)
    pairs_i = _gather_sc(item_r, if2)
    u3 = user.astype(jnp.int32).reshape(B // _DOT_BLK, 1, _DOT_BLK)
    i3 = item.astype(jnp.int32).reshape(B // _DOT_BLK, 1, _DOT_BLK)
    out2 = _dot_tc(u3, i3, pairs_u, pairs_i)
    return out2.reshape(B)


# speedup vs baseline: 1.6460x; 1.0339x over previous
import functools

import jax
import jax.numpy as jnp
from jax import lax
from jax.experimental import pallas as pl
from jax.experimental.pallas import tpu as pltpu
from jax.experimental.pallas import tpu_sc as plsc

B = 16384
F = 64

_info = plsc.get_sparse_core_info()
NC = _info.num_cores
NS = _info.num_subcores
L = _info.num_lanes
NW = NC * NS
BPW = B // NW
CH = 128
NCH = BPW // CH

_mesh = plsc.VectorSubcoreMesh(core_axis_name="c", subcore_axis_name="s")


@functools.partial(
    pl.kernel,
    mesh=_mesh,
    compiler_params=pltpu.CompilerParams(
        needs_layout_passes=False, use_tc_tiling_on_sc=False),
    out_type=jax.ShapeDtypeStruct((B,), jnp.float32),
    scratch_types=[
        pltpu.VMEM((NCH, CH), jnp.int32),
        pltpu.VMEM((NCH, CH), jnp.int32),
        pltpu.VMEM((BPW, F), jnp.float32),
        pltpu.VMEM((BPW, F), jnp.float32),
        pltpu.VMEM((BPW,), jnp.float32),
        pltpu.SemaphoreType.DMA,
    ],
)
def _mf_sc(user_hbm, item_hbm, uf_hbm, if_hbm, out_hbm,
           uidx, iidx, urows, irows, outv, sem):
    wid = lax.axis_index("s") * NC + lax.axis_index("c")

    pltpu.sync_copy(user_hbm.at[wid], uidx)
    pltpu.sync_copy(item_hbm.at[wid], iidx)

    copies = []
    for k in range(NCH):
        copies.append(pltpu.async_copy(
            uf_hbm.at[uidx.at[k]], urows.at[pl.ds(k * CH, CH)], sem))
        copies.append(pltpu.async_copy(
            if_hbm.at[iidx.at[k]], irows.at[pl.ds(k * CH, CH)], sem))
    for c in copies:
        c.wait()

    lanes = lax.iota(jnp.int32, L)

    def group_body(g, _):
        acc = jnp.zeros((L,), jnp.float32)
        for rr in range(L):
            r = g * L + rr
            p = jnp.zeros((L,), jnp.float32)
            for j in range(F // L):
                u = urows[r, pl.ds(j * L, L)]
                v = irows[r, pl.ds(j * L, L)]
                p = p + u * v
            acc = jnp.where(lanes == rr, jnp.sum(p), acc)
        outv[pl.ds(g * L, L)] = acc
        return 0

    lax.fori_loop(0, BPW // L, group_body, 0)

    pltpu.sync_copy(outv, out_hbm.at[pl.ds(wid * BPW, BPW)])


def kernel(user, item, user_factors, item_factors):
    user_r = user.astype(jnp.int32).reshape(NW, NCH, CH)
    item_r = item.astype(jnp.int32).reshape(NW, NCH, CH)
    return _mf_sc(user_r, item_r, user_factors, item_factors)
